# baseline (speedup 1.0000x reference)
import functools

import jax
import jax.numpy as jnp
from jax.experimental import pallas as pl
from jax.experimental.pallas import tpu as pltpu

N_NODES = 62
N_LAYERS = 2
NUM_BRANCHES = 3
TOPK_START = 10
TOPK_END = 3
NEG = -1e30


def _adj_kernel(adj_ref, out_ref, out_t_ref):
    a = adj_ref[...]
    n = a.shape[0]
    rows = jax.lax.broadcasted_iota(jnp.int32, (n, n), 0)
    cols = jax.lax.broadcasted_iota(jnp.int32, (n, n), 1)
    eye = (rows == cols).astype(jnp.float32)
    a = jnp.clip(a, 0.0, 1.0) + eye
    a = jnp.maximum(a, 1e-8)
    at = jnp.transpose(a)
    row_sum = jnp.maximum(jnp.sum(a, axis=1, keepdims=True), 1e-8)
    d = jnp.clip(jax.lax.rsqrt(row_sum), 0.0, 100.0)
    rs_row = jnp.maximum(jnp.sum(at, axis=0, keepdims=True), 1e-8)
    d_row = jnp.clip(jax.lax.rsqrt(rs_row), 0.0, 100.0)
    out_ref[...] = (d * a) * d_row
    out_t_ref[...] = (jnp.transpose(d_row) * at) * jnp.transpose(d)


def _net_kernel(x_ref, adjnt_ref, invt_ref, alpha_ref, out_ref, *, bs, nt):
    n = adjnt_ref.shape[0]
    adjnt = adjnt_ref[...]
    shp = (bs, n, n)
    cols = jax.lax.broadcasted_iota(jnp.int32, shp, 2)
    rows = jax.lax.broadcasted_iota(jnp.int32, shp, 1)
    is_diag = rows == cols
    diag = is_diag.astype(jnp.float32) * 0.1
    off_ok = rows != cols
    adj_dg = jnp.sum(jnp.where(is_diag[:1], adjnt[None], 0.0),
                     axis=1, keepdims=True)

    x0 = [x_ref[:, t] for t in range(nt)]
    cur = x0
    for layer in range(N_LAYERS):
        k_top = int(TOPK_START - (TOPK_START - TOPK_END)
                    * (layer / max(1, N_LAYERS - 1)))
        gram = jnp.stack(
            [sum(jax.lax.dot_general(cur[t][s], cur[t][s],
                                     (((1,), (1,)), ((), ())),
                                     preferred_element_type=jnp.float32)
                 for t in range(nt))
             for s in range(bs)], axis=0)
        g_row = jnp.sum(jnp.where(is_diag, gram, 0.0),
                        axis=1, keepdims=True)
        inv_r = 1.0 / (jnp.sqrt(g_row) + 1e-6)
        inv_outer = jnp.stack(
            [jax.lax.dot_general(inv_r[s], inv_r[s], (((0,), (0,)), ((), ())),
                                 preferred_element_type=jnp.float32)
             for s in range(bs)], axis=0)
        sim = gram * (inv_outer * adjnt[None])
        sim_diag = g_row * inv_r * inv_r * adj_dg
        tmp = jnp.where(off_ok, sim, NEG)
        o_km1 = None
        o_k = None
        for it in range(k_top):
            o_k = jnp.max(tmp, axis=1, keepdims=True)
            if it == k_top - 2:
                o_km1 = o_k
            tmp = jnp.where(tmp >= o_k, NEG, tmp)
        p_acc = None
        for b in range(NUM_BRANCHES):
            invt = invt_ref[b]
            att = sim * invt + diag
            d_b = sim_diag * invt + 0.1
            kth = jnp.minimum(o_km1 * invt,
                              jnp.maximum(o_k * invt, d_b))
            att_m = jnp.where(att >= kth, att, 0.0)
            e = jnp.exp(att_m)
            p = (alpha_ref[layer, b] / jnp.sum(e, axis=1, keepdims=True)) * e
            p_acc = p if p_acc is None else p_acc + p
        new_cur = []
        for t in range(nt):
            y_t = jnp.stack(
                [jax.lax.dot_general(p_acc[s], cur[t][s],
                                     (((0,), (0,)), ((), ())),
                                     preferred_element_type=jnp.float32)
                 for s in range(bs)], axis=0)
            if layer > 0:
                y_t = y_t + x0[t]
            if layer < N_LAYERS - 1:
                y_t = jnp.maximum(y_t, 0.0)
            new_cur.append(y_t)
        cur = new_cur
    c = x_ref.shape[-1]
    for t in range(nt):
        out_ref[:, :, t * c:(t + 1) * c] = cur[t]


@jax.jit
def kernel(x, adj, branch_temps, fusion_logits):
    B, T, N, C = x.shape
    TC = T * C

    adjn, adjnt = pl.pallas_call(
        _adj_kernel,
        out_shape=(jax.ShapeDtypeStruct((N, N), jnp.float32),
                   jax.ShapeDtypeStruct((N, N), jnp.float32)),
    )(adj)

    inv_t = 1.0 / jnp.clip(branch_temps, 0.1, 10.0)
    alpha = jax.nn.softmax(fusion_logits, axis=-1)

    bs = 32
    out = pl.pallas_call(
        functools.partial(_net_kernel, bs=bs, nt=T),
        grid=(B // bs,),
        in_specs=[
            pl.BlockSpec((bs, T, N, C), lambda i: (i, 0, 0, 0)),
            pl.BlockSpec((N, N), lambda i: (0, 0)),
            pl.BlockSpec(memory_space=pltpu.SMEM),
            pl.BlockSpec(memory_space=pltpu.SMEM),
        ],
        out_specs=pl.BlockSpec((bs, N, TC), lambda i: (i, 0, 0)),
        out_shape=jax.ShapeDtypeStruct((B, N, TC), jnp.float32),
    )(x, adjnt, inv_t, alpha)

    return (out, adjn)


# speedup vs baseline: 1.0848x; 1.0848x over previous
import functools

import jax
import jax.numpy as jnp
from jax.experimental import pallas as pl
from jax.experimental.pallas import tpu as pltpu

N_NODES = 62
N_LAYERS = 2
NUM_BRANCHES = 3
TOPK_START = 10
TOPK_END = 3
NEG = -1e30


def _net_kernel(x_ref, adj_ref, bt_ref, fl_ref, out_ref, adjn_ref, *,
                bs, nt):
    n = adj_ref.shape[0]
    rows2 = jax.lax.broadcasted_iota(jnp.int32, (n, n), 0)
    cols2 = jax.lax.broadcasted_iota(jnp.int32, (n, n), 1)
    eye = (rows2 == cols2).astype(jnp.float32)
    a = jnp.clip(adj_ref[...], 0.0, 1.0) + eye
    a = jnp.maximum(a, 1e-8)
    at = jnp.transpose(a)
    row_sum = jnp.maximum(jnp.sum(a, axis=1, keepdims=True), 1e-8)
    d = jnp.clip(jax.lax.rsqrt(row_sum), 0.0, 100.0)
    rs_row = jnp.maximum(jnp.sum(at, axis=0, keepdims=True), 1e-8)
    d_row = jnp.clip(jax.lax.rsqrt(rs_row), 0.0, 100.0)
    adjn_ref[...] = (d * a) * d_row
    adjnt = (jnp.transpose(d_row) * at) * jnp.transpose(d)

    efl = jnp.exp(fl_ref[...])
    alpha = efl / jnp.sum(efl, axis=1, keepdims=True)

    shp = (bs, n, n)
    cols = jax.lax.broadcasted_iota(jnp.int32, shp, 2)
    rows = jax.lax.broadcasted_iota(jnp.int32, shp, 1)
    is_diag = rows == cols
    diag = is_diag.astype(jnp.float32) * 0.1
    off_ok = rows != cols
    adj_dg = jnp.sum(jnp.where(is_diag[:1], adjnt[None], 0.0),
                     axis=1, keepdims=True)

    x0 = [x_ref[:, t] for t in range(nt)]
    cur = x0
    for layer in range(N_LAYERS):
        k_top = int(TOPK_START - (TOPK_START - TOPK_END)
                    * (layer / max(1, N_LAYERS - 1)))
        gram = jnp.stack(
            [sum(jax.lax.dot_general(cur[t][s], cur[t][s],
                                     (((1,), (1,)), ((), ())),
                                     preferred_element_type=jnp.float32)
                 for t in range(nt))
             for s in range(bs)], axis=0)
        g_row = jnp.sum(jnp.where(is_diag, gram, 0.0),
                        axis=1, keepdims=True)
        inv_r = 1.0 / (jnp.sqrt(g_row) + 1e-6)
        inv_outer = jnp.stack(
            [jax.lax.dot_general(inv_r[s], inv_r[s], (((0,), (0,)), ((), ())),
                                 preferred_element_type=jnp.float32)
             for s in range(bs)], axis=0)
        sim = gram * (inv_outer * adjnt[None])
        sim_diag = g_row * inv_r * inv_r * adj_dg
        tmp = jnp.where(off_ok, sim, NEG)
        o_km1 = None
        o_k = None
        for it in range(k_top):
            o_k = jnp.max(tmp, axis=1, keepdims=True)
            if it == k_top - 2:
                o_km1 = o_k
            tmp = jnp.where(tmp >= o_k, NEG, tmp)
        p_acc = None
        for b in range(NUM_BRANCHES):
            invt = 1.0 / jnp.clip(bt_ref[b], 0.1, 10.0)
            att = sim * invt + diag
            d_b = sim_diag * invt + 0.1
            kth = jnp.minimum(o_km1 * invt,
                              jnp.maximum(o_k * invt, d_b))
            att_m = jnp.where(att >= kth, att, 0.0)
            e = jnp.exp(att_m)
            al = alpha[layer:layer + 1, b:b + 1].reshape(1, 1, 1)
            p = (al / jnp.sum(e, axis=1, keepdims=True)) * e
            p_acc = p if p_acc is None else p_acc + p
        new_cur = []
        for t in range(nt):
            y_t = jnp.stack(
                [jax.lax.dot_general(p_acc[s], cur[t][s],
                                     (((0,), (0,)), ((), ())),
                                     preferred_element_type=jnp.float32)
                 for s in range(bs)], axis=0)
            if layer > 0:
                y_t = y_t + x0[t]
            if layer < N_LAYERS - 1:
                y_t = jnp.maximum(y_t, 0.0)
            new_cur.append(y_t)
        cur = new_cur
    c = x_ref.shape[-1]
    for t in range(nt):
        out_ref[:, :, t * c:(t + 1) * c] = cur[t]


@jax.jit
def kernel(x, adj, branch_temps, fusion_logits):
    B, T, N, C = x.shape
    TC = T * C

    bs = 16
    out, adjn = pl.pallas_call(
        functools.partial(_net_kernel, bs=bs, nt=T),
        grid=(B // bs,),
        in_specs=[
            pl.BlockSpec((bs, T, N, C), lambda i: (i, 0, 0, 0)),
            pl.BlockSpec((N, N), lambda i: (0, 0)),
            pl.BlockSpec(memory_space=pltpu.SMEM),
            pl.BlockSpec((N_LAYERS, NUM_BRANCHES), lambda i: (0, 0)),
        ],
        out_specs=(pl.BlockSpec((bs, N, TC), lambda i: (i, 0, 0)),
                   pl.BlockSpec((N, N), lambda i: (0, 0))),
        out_shape=(jax.ShapeDtypeStruct((B, N, TC), jnp.float32),
                   jax.ShapeDtypeStruct((N, N), jnp.float32)),
    )(x, adj, branch_temps, fusion_logits)

    return (out, adjn)
